# baseline (speedup 1.0000x reference)
import jax
import jax.numpy as jnp
from jax.experimental import pallas as pl
from jax.experimental.pallas import tpu as pltpu
from jax.experimental.pallas import tpu_sc as plsc

_N_BLK = 1024
_PACK = 4


def _gather_rows128(table128, u):
    b = u.shape[0]
    n_workers = 32
    b_per_w = b // n_workers
    mesh = plsc.VectorSubcoreMesh(core_axis_name="c", subcore_axis_name="s")
    q = (u // _PACK).astype(jnp.int32)

    @pl.kernel(out_type=jax.ShapeDtypeStruct((b, 128), table128.dtype),
               mesh=mesh,
               scratch_types=[
                   pltpu.VMEM((b_per_w,), jnp.int32),
                   pltpu.VMEM((b_per_w, 128), jnp.float32),
                   pltpu.SemaphoreType.DMA,
               ])
    def gather_kernel(table_hbm, idx_hbm, out_hbm, idx_v, rows_v, sem):
        wid = jax.lax.axis_index("s") * 2 + jax.lax.axis_index("c")
        base = wid * b_per_w
        pltpu.sync_copy(idx_hbm.at[pl.ds(base, b_per_w)], idx_v)
        pltpu.async_copy(table_hbm.at[idx_v], rows_v, sem).wait()
        pltpu.sync_copy(rows_v, out_hbm.at[pl.ds(base, b_per_w)])

    return gather_kernel(table128, q)


def _score_kernel(rows_ref, r_ref, v_ref, o_ref, user_scratch):
    @pl.when(pl.program_id(0) == 0)
    def _():
        r = r_ref[...]
        rows = rows_ref[...]
        dim = rows.shape[1] // _PACK
        user = rows[:, 3 * dim:]
        for k in range(_PACK - 2, -1, -1):
            user = jnp.where(r == k, rows[:, k * dim:(k + 1) * dim], user)
        user_scratch[...] = user.astype(jnp.bfloat16)

    s = jax.lax.dot_general(
        user_scratch[...], v_ref[...],
        dimension_numbers=(((1,), (1,)), ((), ())),
        preferred_element_type=jnp.float32)
    o_ref[...] = jax.nn.sigmoid(s)


def kernel(u, U_emb, V_emb):
    b = u.shape[0]
    n_user, dim = U_emb.shape
    n_item = V_emb.shape[0]
    table128 = U_emb.reshape(n_user * dim // 128, 128)
    rows128 = _gather_rows128(table128, u)
    r = (u % _PACK).astype(jnp.int32).reshape(b, 1)
    v16 = V_emb.astype(jnp.bfloat16)
    return pl.pallas_call(
        _score_kernel,
        grid=(pl.cdiv(n_item, _N_BLK),),
        in_specs=[
            pl.BlockSpec((b, 128), lambda i: (0, 0)),
            pl.BlockSpec((b, 1), lambda i: (0, 0)),
            pl.BlockSpec((_N_BLK, dim), lambda i: (i, 0)),
        ],
        out_specs=pl.BlockSpec((b, _N_BLK), lambda i: (0, i)),
        out_shape=jax.ShapeDtypeStruct((b, n_item), jnp.float32),
        scratch_shapes=[pltpu.VMEM((b, dim), jnp.bfloat16)],
    )(rows128, r, v16)


# speedup vs baseline: 1.0064x; 1.0064x over previous
import jax
import jax.numpy as jnp
from jax.experimental import pallas as pl
from jax.experimental.pallas import tpu as pltpu
from jax.experimental.pallas import tpu_sc as plsc

_N_BLK = 1024
_PACK = 4


def _gather_rows128(table128, u):
    b = u.shape[0]
    n_workers = 32
    b_per_w = b // n_workers
    mesh = plsc.VectorSubcoreMesh(core_axis_name="c", subcore_axis_name="s")
    q = (u // _PACK).astype(jnp.int32)

    @pl.kernel(out_type=jax.ShapeDtypeStruct((b, 128), table128.dtype),
               mesh=mesh,
               scratch_types=[
                   pltpu.VMEM((b_per_w,), jnp.int32),
                   pltpu.VMEM((b_per_w, 128), jnp.float32),
                   pltpu.SemaphoreType.DMA,
               ])
    def gather_kernel(table_hbm, idx_hbm, out_hbm, idx_v, rows_v, sem):
        wid = jax.lax.axis_index("s") * 2 + jax.lax.axis_index("c")
        base = wid * b_per_w
        pltpu.sync_copy(idx_hbm.at[pl.ds(base, b_per_w)], idx_v)
        pltpu.async_copy(table_hbm.at[idx_v], rows_v, sem).wait()
        pltpu.sync_copy(rows_v, out_hbm.at[pl.ds(base, b_per_w)])

    return gather_kernel(table128, q)


def _score_kernel(rows_ref, r_ref, v_ref, o_ref, user_scratch):
    @pl.when(pl.program_id(0) == 0)
    def _():
        r = r_ref[...]
        rows = rows_ref[...]
        dim = rows.shape[1] // _PACK
        user = rows[:, 3 * dim:]
        for k in range(_PACK - 2, -1, -1):
            user = jnp.where(r == k, rows[:, k * dim:(k + 1) * dim], user)
        user_scratch[...] = user.astype(jnp.bfloat16)

    s = jax.lax.dot_general(
        user_scratch[...], v_ref[...],
        dimension_numbers=(((1,), (1,)), ((), ())),
        preferred_element_type=jnp.float32)
    o_ref[...] = jax.nn.sigmoid(s)


def kernel(u, U_emb, V_emb):
    b = u.shape[0]
    n_user, dim = U_emb.shape
    n_item = V_emb.shape[0]
    table128 = U_emb.reshape(n_user * dim // 128, 128)
    rows128 = jnp.take(table128, (u // _PACK).astype(jnp.int32), axis=0)
    r = (u % _PACK).astype(jnp.int32).reshape(b, 1)
    v16 = V_emb.astype(jnp.bfloat16)
    return pl.pallas_call(
        _score_kernel,
        grid=(pl.cdiv(n_item, _N_BLK),),
        in_specs=[
            pl.BlockSpec((b, 128), lambda i: (0, 0)),
            pl.BlockSpec((b, 1), lambda i: (0, 0)),
            pl.BlockSpec((_N_BLK, dim), lambda i: (i, 0)),
        ],
        out_specs=pl.BlockSpec((b, _N_BLK), lambda i: (0, i)),
        out_shape=jax.ShapeDtypeStruct((b, n_item), jnp.float32),
        scratch_shapes=[pltpu.VMEM((b, dim), jnp.bfloat16)],
    )(rows128, r, v16)


# speedup vs baseline: 4.1895x; 4.1629x over previous
import dataclasses

import jax
import jax.numpy as jnp
from jax.experimental import pallas as pl
from jax.experimental.pallas import tpu as pltpu
from jax.experimental.pallas import tpu_sc as plsc

_N_BLK = 1024


def _gather_cols(ut, u):
    dim, _ = ut.shape
    b = u.shape[0]
    n_workers = 32
    bpw = b // n_workers
    slots = 16
    mesh = plsc.VectorSubcoreMesh(core_axis_name="c", subcore_axis_name="s")
    cp = pltpu.CompilerParams()
    if "needs_layout_passes" in pltpu.CompilerParams.__dataclass_fields__:
        cp = dataclasses.replace(cp, needs_layout_passes=False)

    @pl.kernel(out_type=jax.ShapeDtypeStruct((dim, b), ut.dtype),
               mesh=mesh,
               compiler_params=cp,
               scratch_types=[
                   pltpu.VMEM((bpw,), jnp.int32),
                   pltpu.VMEM((slots, dim, 128), jnp.float32),
                   pltpu.VMEM((dim, bpw), jnp.float32),
                   pltpu.SemaphoreType.DMA,
               ])
    def gather_kernel(ut_hbm, idx_hbm, out_hbm, idx_v, stage_v,
                      cols_v, sem):
        wid = jax.lax.axis_index("s") * 2 + jax.lax.axis_index("c")
        base = pl.multiple_of(wid * bpw, bpw)
        pltpu.sync_copy(idx_hbm.at[pl.ds(base, bpw)], idx_v)

        lanes16 = jax.lax.iota(jnp.int32, 16)
        rows_hi = lanes16 + 16
        ones = jnp.ones((16,), jnp.int32)

        @pl.loop(0, bpw // slots)
        def _(c):
            off = pl.multiple_of(c * slots, slots)
            idx_reg = idx_v[pl.ds(off, slots)]

            @pl.loop(0, slots)
            def _(j):
                ui = jnp.sum(jnp.where(lanes16 == j, idx_reg, 0))
                a = pl.multiple_of(ui - ui % 128, 128)
                pltpu.make_async_copy(ut_hbm.at[:, pl.ds(a, 128)],
                                      stage_v.at[j], sem).start()

            @pl.loop(0, slots)
            def _(j):
                pltpu.make_async_copy(ut_hbm.at[:, pl.ds(0, 128)],
                                      stage_v.at[j], sem).wait()
                i = c * slots + j
                lane = plsc.load_gather(idx_v, [i * ones]) % 128
                slot = j * ones
                col = i * ones
                v_lo = plsc.load_gather(stage_v, [slot, lanes16, lane])
                v_hi = plsc.load_gather(stage_v, [slot, rows_hi, lane])
                plsc.store_scatter(cols_v, [lanes16, col], v_lo)
                plsc.store_scatter(cols_v, [rows_hi, col], v_hi)

        pltpu.sync_copy(cols_v, out_hbm.at[:, pl.ds(base, bpw)])

    return gather_kernel(ut, u.astype(jnp.int32))


def _score_kernel(vt_ref, ut_ref, o_ref):
    s = jax.lax.dot_general(
        vt_ref[...], ut_ref[...],
        dimension_numbers=(((0,), (0,)), ((), ())),
        preferred_element_type=jnp.float32)
    o_ref[...] = jax.nn.sigmoid(s)


def kernel(u, U_emb, V_emb):
    b = u.shape[0]
    dim = U_emb.shape[1]
    n_item = V_emb.shape[0]
    ut = U_emb.T
    userT = _gather_cols(ut, u)
    userT16 = userT.astype(jnp.bfloat16)
    vt16 = V_emb.T.astype(jnp.bfloat16)
    scoresT = pl.pallas_call(
        _score_kernel,
        grid=(pl.cdiv(n_item, _N_BLK),),
        in_specs=[
            pl.BlockSpec((dim, _N_BLK), lambda i: (0, i)),
            pl.BlockSpec((dim, b), lambda i: (0, 0)),
        ],
        out_specs=pl.BlockSpec((_N_BLK, b), lambda i: (i, 0)),
        out_shape=jax.ShapeDtypeStruct((n_item, b), jnp.float32),
    )(vt16, userT16)
    return scoresT.T


# speedup vs baseline: 4.5429x; 1.0844x over previous
import dataclasses

import jax
import jax.numpy as jnp
from jax.experimental import pallas as pl
from jax.experimental.pallas import tpu as pltpu
from jax.experimental.pallas import tpu_sc as plsc

_N_BLK = 1024


def _gather_cols(ut, u):
    dim, _ = ut.shape
    b = u.shape[0]
    n_workers = 32
    bpw = b // n_workers
    slots = 16
    mesh = plsc.VectorSubcoreMesh(core_axis_name="c", subcore_axis_name="s")
    cp = pltpu.CompilerParams()
    if "needs_layout_passes" in pltpu.CompilerParams.__dataclass_fields__:
        cp = dataclasses.replace(cp, needs_layout_passes=False)

    @pl.kernel(out_type=jax.ShapeDtypeStruct((dim, b), ut.dtype),
               mesh=mesh,
               compiler_params=cp,
               scratch_types=[
                   pltpu.VMEM((bpw,), jnp.int32),
                   pltpu.VMEM((slots, dim, 128), jnp.float32),
                   pltpu.VMEM((dim, bpw), jnp.float32),
                   pltpu.SemaphoreType.DMA,
               ])
    def gather_kernel(ut_hbm, idx_hbm, out_hbm, idx_v, stage_v,
                      cols_v, sem):
        wid = jax.lax.axis_index("s") * 2 + jax.lax.axis_index("c")
        base = pl.multiple_of(wid * bpw, bpw)
        pltpu.sync_copy(idx_hbm.at[pl.ds(base, bpw)], idx_v)

        lanes16 = jax.lax.iota(jnp.int32, 16)
        rows_hi = lanes16 + 16
        ones = jnp.ones((16,), jnp.int32)

        @pl.loop(0, bpw // slots)
        def _(c):
            off = pl.multiple_of(c * slots, slots)
            idx_reg = idx_v[pl.ds(off, slots)]

            @pl.loop(0, slots)
            def _(j):
                ui = jnp.sum(jnp.where(lanes16 == j, idx_reg, 0))
                a = pl.multiple_of(ui - ui % 128, 128)
                pltpu.make_async_copy(ut_hbm.at[:, pl.ds(a, 128)],
                                      stage_v.at[j], sem).start()

            @pl.loop(0, slots)
            def _(j):
                pltpu.make_async_copy(ut_hbm.at[:, pl.ds(0, 128)],
                                      stage_v.at[j], sem).wait()
                i = c * slots + j
                lane = plsc.load_gather(idx_v, [i * ones]) % 128
                slot = j * ones
                col = i * ones
                v_lo = plsc.load_gather(stage_v, [slot, lanes16, lane])
                v_hi = plsc.load_gather(stage_v, [slot, rows_hi, lane])
                plsc.store_scatter(cols_v, [lanes16, col], v_lo)
                plsc.store_scatter(cols_v, [rows_hi, col], v_hi)

        pltpu.sync_copy(cols_v, out_hbm.at[:, pl.ds(base, bpw)])

    return gather_kernel(ut, u.astype(jnp.int32))


def _score_kernel(vt_ref, ut_ref, o_ref):
    s = jax.lax.dot_general(
        vt_ref[...], ut_ref[...],
        dimension_numbers=(((0,), (0,)), ((), ())),
        preferred_element_type=jnp.float32)
    o_ref[...] = 0.5 + 0.5 * jnp.tanh(0.5 * s)


def kernel(u, U_emb, V_emb):
    b = u.shape[0]
    dim = U_emb.shape[1]
    n_item = V_emb.shape[0]
    ut = U_emb.T
    userT = _gather_cols(ut, u)
    userT16 = userT.astype(jnp.bfloat16)
    vt16 = V_emb.T.astype(jnp.bfloat16)
    scoresT = pl.pallas_call(
        _score_kernel,
        grid=(pl.cdiv(n_item, _N_BLK),),
        in_specs=[
            pl.BlockSpec((dim, _N_BLK), lambda i: (0, i)),
            pl.BlockSpec((dim, b), lambda i: (0, 0)),
        ],
        out_specs=pl.BlockSpec((_N_BLK, b), lambda i: (i, 0)),
        out_shape=jax.ShapeDtypeStruct((n_item, b), jnp.float32),
    )(vt16, userT16)
    return scoresT.T


# speedup vs baseline: 4.5523x; 1.0021x over previous
import dataclasses

import jax
import jax.numpy as jnp
from jax.experimental import pallas as pl
from jax.experimental.pallas import tpu as pltpu
from jax.experimental.pallas import tpu_sc as plsc

_N_BLK = 1536


def _gather_cols(ut, u):
    dim, _ = ut.shape
    b = u.shape[0]
    n_workers = 32
    bpw = b // n_workers
    slots = 16
    mesh = plsc.VectorSubcoreMesh(core_axis_name="c", subcore_axis_name="s")
    cp = pltpu.CompilerParams()
    if "needs_layout_passes" in pltpu.CompilerParams.__dataclass_fields__:
        cp = dataclasses.replace(cp, needs_layout_passes=False)

    @pl.kernel(out_type=jax.ShapeDtypeStruct((dim, b), ut.dtype),
               mesh=mesh,
               compiler_params=cp,
               scratch_types=[
                   pltpu.VMEM((bpw,), jnp.int32),
                   pltpu.VMEM((slots, dim, 128), jnp.float32),
                   pltpu.VMEM((dim, bpw), jnp.float32),
                   pltpu.SemaphoreType.DMA,
               ])
    def gather_kernel(ut_hbm, idx_hbm, out_hbm, idx_v, stage_v,
                      cols_v, sem):
        wid = jax.lax.axis_index("s") * 2 + jax.lax.axis_index("c")
        base = pl.multiple_of(wid * bpw, bpw)
        pltpu.sync_copy(idx_hbm.at[pl.ds(base, bpw)], idx_v)

        lanes16 = jax.lax.iota(jnp.int32, 16)
        rows_hi = lanes16 + 16
        ones = jnp.ones((16,), jnp.int32)

        @pl.loop(0, bpw // slots)
        def _(c):
            off = pl.multiple_of(c * slots, slots)
            idx_reg = idx_v[pl.ds(off, slots)]

            @pl.loop(0, slots)
            def _(j):
                ui = jnp.sum(jnp.where(lanes16 == j, idx_reg, 0))
                a = pl.multiple_of(ui - ui % 128, 128)
                pltpu.make_async_copy(ut_hbm.at[:, pl.ds(a, 128)],
                                      stage_v.at[j], sem).start()

            @pl.loop(0, slots)
            def _(j):
                pltpu.make_async_copy(ut_hbm.at[:, pl.ds(0, 128)],
                                      stage_v.at[j], sem).wait()
                i = c * slots + j
                lane = plsc.load_gather(idx_v, [i * ones]) % 128
                slot = j * ones
                col = i * ones
                v_lo = plsc.load_gather(stage_v, [slot, lanes16, lane])
                v_hi = plsc.load_gather(stage_v, [slot, rows_hi, lane])
                plsc.store_scatter(cols_v, [lanes16, col], v_lo)
                plsc.store_scatter(cols_v, [rows_hi, col], v_hi)

        pltpu.sync_copy(cols_v, out_hbm.at[:, pl.ds(base, bpw)])

    return gather_kernel(ut, u.astype(jnp.int32))


def _score_kernel(vt_ref, ut_ref, o_ref):
    s = jax.lax.dot_general(
        vt_ref[...], ut_ref[...],
        dimension_numbers=(((0,), (0,)), ((), ())),
        preferred_element_type=jnp.float32)
    o_ref[...] = 0.5 + 0.5 * jnp.tanh(0.5 * s)


def kernel(u, U_emb, V_emb):
    b = u.shape[0]
    dim = U_emb.shape[1]
    n_item = V_emb.shape[0]
    ut = U_emb.T
    userT = _gather_cols(ut, u)
    userT16 = userT.astype(jnp.bfloat16)
    vt16 = V_emb.T.astype(jnp.bfloat16)
    scoresT = pl.pallas_call(
        _score_kernel,
        grid=(pl.cdiv(n_item, _N_BLK),),
        in_specs=[
            pl.BlockSpec((dim, _N_BLK), lambda i: (0, i)),
            pl.BlockSpec((dim, b), lambda i: (0, 0)),
        ],
        out_specs=pl.BlockSpec((_N_BLK, b), lambda i: (i, 0)),
        out_shape=jax.ShapeDtypeStruct((n_item, b), jnp.float32),
    )(vt16, userT16)
    return scoresT.T


# speedup vs baseline: 4.5532x; 1.0002x over previous
import dataclasses

import jax
import jax.numpy as jnp
from jax.experimental import pallas as pl
from jax.experimental.pallas import tpu as pltpu
from jax.experimental.pallas import tpu_sc as plsc

_N_BLK = 1664


def _gather_cols(ut, u):
    dim, _ = ut.shape
    b = u.shape[0]
    n_workers = 32
    bpw = b // n_workers
    slots = 16
    mesh = plsc.VectorSubcoreMesh(core_axis_name="c", subcore_axis_name="s")
    cp = pltpu.CompilerParams()
    if "needs_layout_passes" in pltpu.CompilerParams.__dataclass_fields__:
        cp = dataclasses.replace(cp, needs_layout_passes=False)

    @pl.kernel(out_type=jax.ShapeDtypeStruct((dim, b), ut.dtype),
               mesh=mesh,
               compiler_params=cp,
               scratch_types=[
                   pltpu.VMEM((bpw,), jnp.int32),
                   pltpu.VMEM((slots, dim, 128), jnp.float32),
                   pltpu.VMEM((dim, bpw), jnp.float32),
                   pltpu.SemaphoreType.DMA,
               ])
    def gather_kernel(ut_hbm, idx_hbm, out_hbm, idx_v, stage_v,
                      cols_v, sem):
        wid = jax.lax.axis_index("s") * 2 + jax.lax.axis_index("c")
        base = pl.multiple_of(wid * bpw, bpw)
        pltpu.sync_copy(idx_hbm.at[pl.ds(base, bpw)], idx_v)

        lanes16 = jax.lax.iota(jnp.int32, 16)
        rows_hi = lanes16 + 16
        ones = jnp.ones((16,), jnp.int32)

        @pl.loop(0, bpw // slots)
        def _(c):
            off = pl.multiple_of(c * slots, slots)
            idx_reg = idx_v[pl.ds(off, slots)]

            @pl.loop(0, slots)
            def _(j):
                ui = jnp.sum(jnp.where(lanes16 == j, idx_reg, 0))
                a = pl.multiple_of(ui - ui % 128, 128)
                pltpu.make_async_copy(ut_hbm.at[:, pl.ds(a, 128)],
                                      stage_v.at[j], sem).start()

            @pl.loop(0, slots)
            def _(j):
                pltpu.make_async_copy(ut_hbm.at[:, pl.ds(0, 128)],
                                      stage_v.at[j], sem).wait()
                i = c * slots + j
                lane = plsc.load_gather(idx_v, [i * ones]) % 128
                slot = j * ones
                col = i * ones
                v_lo = plsc.load_gather(stage_v, [slot, lanes16, lane])
                v_hi = plsc.load_gather(stage_v, [slot, rows_hi, lane])
                plsc.store_scatter(cols_v, [lanes16, col], v_lo)
                plsc.store_scatter(cols_v, [rows_hi, col], v_hi)

        pltpu.sync_copy(cols_v, out_hbm.at[:, pl.ds(base, bpw)])

    return gather_kernel(ut, u.astype(jnp.int32))


def _score_kernel(vt_ref, ut_ref, o_ref):
    s = jax.lax.dot_general(
        vt_ref[...], ut_ref[...],
        dimension_numbers=(((0,), (0,)), ((), ())),
        preferred_element_type=jnp.float32)
    o_ref[...] = 0.5 + 0.5 * jnp.tanh(s)


def kernel(u, U_emb, V_emb):
    b = u.shape[0]
    dim = U_emb.shape[1]
    n_item = V_emb.shape[0]
    ut = U_emb.T
    userT = _gather_cols(ut, u)
    userT16 = userT.astype(jnp.bfloat16)
    vt16 = (0.5 * V_emb).T.astype(jnp.bfloat16)
    scoresT = pl.pallas_call(
        _score_kernel,
        grid=(pl.cdiv(n_item, _N_BLK),),
        in_specs=[
            pl.BlockSpec((dim, _N_BLK), lambda i: (0, i)),
            pl.BlockSpec((dim, b), lambda i: (0, 0)),
        ],
        out_specs=pl.BlockSpec((_N_BLK, b), lambda i: (i, 0)),
        out_shape=jax.ShapeDtypeStruct((n_item, b), jnp.float32),
    )(vt16, userT16)
    return scoresT.T


# speedup vs baseline: 4.5848x; 1.0069x over previous
import dataclasses

import jax
import jax.numpy as jnp
from jax.experimental import pallas as pl
from jax.experimental.pallas import tpu as pltpu
from jax.experimental.pallas import tpu_sc as plsc

_N_BLK = 1664


def _gather_cols(ut, u):
    dim, _ = ut.shape
    b = u.shape[0]
    n_workers = 32
    bpw = b // n_workers
    slots = 16
    mesh = plsc.VectorSubcoreMesh(core_axis_name="c", subcore_axis_name="s")
    cp = pltpu.CompilerParams()
    if "needs_layout_passes" in pltpu.CompilerParams.__dataclass_fields__:
        cp = dataclasses.replace(cp, needs_layout_passes=False)

    @pl.kernel(out_type=jax.ShapeDtypeStruct((dim, b), ut.dtype),
               mesh=mesh,
               compiler_params=cp,
               scratch_types=[
                   pltpu.VMEM((bpw,), jnp.int32),
                   pltpu.VMEM((slots, dim, 128), jnp.float32),
                   pltpu.VMEM((dim, bpw), jnp.float32),
                   pltpu.SemaphoreType.DMA,
               ])
    def gather_kernel(ut_hbm, idx_hbm, out_hbm, idx_v, stage_v,
                      cols_v, sem):
        wid = jax.lax.axis_index("s") * 2 + jax.lax.axis_index("c")
        base = pl.multiple_of(wid * bpw, bpw)
        pltpu.sync_copy(idx_hbm.at[pl.ds(base, bpw)], idx_v)

        lanes16 = jax.lax.iota(jnp.int32, 16)
        rows_hi = lanes16 + 16
        ones = jnp.ones((16,), jnp.int32)

        def fire(i, slot_id):
            ui = jnp.max(plsc.load_gather(idx_v, [i * ones]))
            a = pl.multiple_of(ui - ui % 128, 128)
            pltpu.make_async_copy(ut_hbm.at[:, pl.ds(a, 128)],
                                  stage_v.at[slot_id], sem).start()

        @pl.loop(0, slots)
        def _(j):
            fire(j, j)

        @pl.loop(0, bpw)
        def _(i):
            j = i % slots
            pltpu.make_async_copy(ut_hbm.at[:, pl.ds(0, 128)],
                                  stage_v.at[j], sem).wait()
            lane = plsc.load_gather(idx_v, [i * ones]) % 128
            slot = j * ones
            col = i * ones
            v_lo = plsc.load_gather(stage_v, [slot, lanes16, lane])
            v_hi = plsc.load_gather(stage_v, [slot, rows_hi, lane])
            plsc.store_scatter(cols_v, [lanes16, col], v_lo)
            plsc.store_scatter(cols_v, [rows_hi, col], v_hi)

            @pl.when(i + slots < bpw)
            def _():
                fire(i + slots, j)

        pltpu.sync_copy(cols_v, out_hbm.at[:, pl.ds(base, bpw)])

    return gather_kernel(ut, u.astype(jnp.int32))


def _score_kernel(vt_ref, ut_ref, o_ref):
    s = jax.lax.dot_general(
        vt_ref[...], ut_ref[...],
        dimension_numbers=(((0,), (0,)), ((), ())),
        preferred_element_type=jnp.float32)
    o_ref[...] = 0.5 + 0.5 * jnp.tanh(s)


def kernel(u, U_emb, V_emb):
    b = u.shape[0]
    dim = U_emb.shape[1]
    n_item = V_emb.shape[0]
    ut = U_emb.T
    userT = _gather_cols(ut, u)
    userT16 = userT.astype(jnp.bfloat16)
    vt16 = (0.5 * V_emb).T.astype(jnp.bfloat16)
    scoresT = pl.pallas_call(
        _score_kernel,
        grid=(pl.cdiv(n_item, _N_BLK),),
        in_specs=[
            pl.BlockSpec((dim, _N_BLK), lambda i: (0, i)),
            pl.BlockSpec((dim, b), lambda i: (0, 0)),
        ],
        out_specs=pl.BlockSpec((_N_BLK, b), lambda i: (i, 0)),
        out_shape=jax.ShapeDtypeStruct((n_item, b), jnp.float32),
    )(vt16, userT16)
    return scoresT.T
